# Initial kernel scaffold; baseline (speedup 1.0000x reference)
import jax
import jax.numpy as jnp
from jax.experimental import pallas as pl


def kernel(x, edge_index, edge_weight, batch, W, b, prelu_a, gamma, beta):
    raise NotImplementedError("write your pallas kernel here")


# speedup vs baseline: 14.3942x; 14.3942x over previous
import functools

import jax
import jax.numpy as jnp
from jax import lax
from jax.experimental import pallas as pl
from jax.experimental.pallas import tpu as pltpu
from jax.experimental.pallas import tpu_sc as plsc

G = 16
K = 64
SUP = 8
NC = 2
NS = 16
NW = NC * NS


def _bcast_lane(vec16, l):
    idx = jnp.full((16,), l, dtype=jnp.int32)
    return lax.gather(
        vec16,
        idx[:, None],
        dimension_numbers=lax.GatherDimensionNumbers(
            offset_dims=(), collapsed_slice_dims=(0,), start_index_map=(0,)),
        slice_sizes=(1,),
        mode=lax.GatherScatterMode.PROMISE_IN_BOUNDS)


def _zero_buf(buf, rows, cols):
    z = jnp.zeros((16,), jnp.float32)

    def body(r, _):
        for v in range(cols // 16):
            buf[r, pl.ds(v * 16, 16)] = z
        return 0

    lax.fori_loop(0, rows, body, 0)


def _make_sc_deg(n_pad, c0, c1):
    rows_per_tile = n_pad // NS
    cmax = max(c0, c1)
    mesh = plsc.VectorSubcoreMesh(core_axis_name="c", subcore_axis_name="s")

    @functools.partial(
        pl.kernel,
        mesh=mesh,
        out_type=jax.ShapeDtypeStruct((NC, n_pad), jnp.float32),
        scratch_types=[
            pltpu.VMEM((cmax, K), jnp.int32),
            pltpu.VMEM((cmax, K), jnp.float32),
            pltpu.VMEM((rows_per_tile,), jnp.float32),
            pltpu.VMEM_SHARED((n_pad,), jnp.float32),
            pltpu.SemaphoreType.DMA,
        ],
    )
    def deg_kernel(col_hbm, w_hbm, out_hbm, col_v, w_v, zbuf, acc, sem):
        cid = lax.axis_index("c")
        sid = lax.axis_index("s")

        z = jnp.zeros((16,), jnp.float32)

        def zb(t, _):
            zbuf[pl.ds(t * 16, 16)] = z
            return 0

        lax.fori_loop(0, rows_per_tile // 16, zb, 0)
        pltpu.sync_copy(zbuf, acc.at[pl.ds(sid * rows_per_tile,
                                           rows_per_tile)])
        plsc.subcore_barrier()

        base = sid * (c0 + c1) + cid * c0
        nch = jnp.where(cid == 0, c0, c1)

        @pl.when(cid == 0)
        def _():
            pltpu.sync_copy(col_hbm.at[pl.ds(base, c0)],
                            col_v.at[pl.ds(0, c0)])
            pltpu.sync_copy(w_hbm.at[pl.ds(base, c0)], w_v.at[pl.ds(0, c0)])

        @pl.when(cid == 1)
        def _():
            pltpu.sync_copy(col_hbm.at[pl.ds(base, c1)],
                            col_v.at[pl.ds(0, c1)])
            pltpu.sync_copy(w_hbm.at[pl.ds(base, c1)], w_v.at[pl.ds(0, c1)])

        depth = min(16, c0, c1)
        nchunk = nch

        def fire(j):
            pltpu.async_copy(w_v.at[j], acc.at[col_v.at[j]], sem, add=True)

        def wait_one():
            pltpu.make_async_copy(w_v.at[0], acc.at[col_v.at[0]], sem).wait()

        def fire_body(j, _):
            fire(j)
            return 0

        lax.fori_loop(0, depth, fire_body, 0)

        def roll_body(j, _):
            wait_one()
            fire(j)
            return 0

        lax.fori_loop(depth, nchunk, roll_body, 0)

        def drain_body(j, _):
            wait_one()
            return 0

        lax.fori_loop(0, depth, drain_body, 0)

        plsc.subcore_barrier()
        pltpu.sync_copy(
            acc.at[pl.ds(sid * rows_per_tile, rows_per_tile)],
            out_hbm.at[cid, pl.ds(sid * rows_per_tile, rows_per_tile)])

    return deg_kernel


def _make_sc_hop(n_pad, c, n0, n1):
    rows_per_tile = n_pad // NS
    nblk = n_pad // K
    mesh = plsc.VectorSubcoreMesh(core_axis_name="c", subcore_axis_name="s")

    @functools.partial(
        pl.kernel,
        mesh=mesh,
        out_type=jax.ShapeDtypeStruct((NC, n_pad, c), jnp.float32),
        compiler_params=pltpu.CompilerParams(use_tc_tiling_on_sc=False),
        scratch_types=[
            pltpu.VMEM((2, SUP, K), jnp.int32),
            pltpu.VMEM((2, SUP, K), jnp.int32),
            pltpu.VMEM((2, SUP, K), jnp.float32),
            pltpu.VMEM((K, c // 2), jnp.int32),
            pltpu.VMEM((K, c // 2), jnp.int32),
            pltpu.VMEM((K, c), jnp.float32),
            pltpu.VMEM((K, c), jnp.float32),
            pltpu.VMEM_SHARED((n_pad, c), jnp.float32),
            pltpu.SemaphoreType.DMA,
            pltpu.SemaphoreType.DMA,
            pltpu.SemaphoreType.DMA,
            pltpu.SemaphoreType.DMA,
            pltpu.SemaphoreType.DMA,
            pltpu.SemaphoreType.DMA,
        ],
    )
    def hop_kernel(g_hbm, row_hbm, col_hbm, w_hbm, out_hbm,
                   row_r, col_r, w_r, gbuf0, gbuf1, sbuf0, sbuf1,
                   acc, isem0, isem1, gsem0, gsem1, ssem0, ssem1):
        cid = lax.axis_index("c")
        sid = lax.axis_index("s")

        _zero_buf(sbuf0, K, c)
        for t in range(-(-nblk // NS)):
            blk_id = sid + t * NS

            @pl.when(blk_id < nblk)
            def _():
                pltpu.sync_copy(sbuf0, acc.at[pl.ds(blk_id * K, K)])
        plsc.subcore_barrier()

        sbase = sid * (n0 + n1) + cid * n0
        nsuper = jnp.where(cid == 0, n0, n1)
        isems = (isem0, isem1)
        gbufs = ((gbuf0, gsem0), (gbuf1, gsem1))
        sbufs = ((sbuf0, ssem0), (sbuf1, ssem1))

        def stage(s, slot):
            pltpu.async_copy(row_hbm.at[sbase + s], row_r.at[slot],
                             isems[slot])
            pltpu.async_copy(col_hbm.at[sbase + s], col_r.at[slot],
                             isems[slot])
            pltpu.async_copy(w_hbm.at[sbase + s], w_r.at[slot], isems[slot])

        def wait_stage(slot):
            pltpu.make_async_copy(row_hbm.at[sbase], row_r.at[slot],
                                  isems[slot]).wait()
            pltpu.make_async_copy(col_hbm.at[sbase], col_r.at[slot],
                                  isems[slot]).wait()
            pltpu.make_async_copy(w_hbm.at[sbase], w_r.at[slot],
                                  isems[slot]).wait()

        himask = jnp.full((16,), -65536, dtype=jnp.int32)

        def scale_rows(slot, q, src, dst):
            def gi_body(gi, _):
                wv = w_r[slot, q, pl.ds(gi * 16, 16)]
                for l in range(16):
                    bc = _bcast_lane(wv, l)
                    r = gi * 16 + l
                    for v in range(c // 32):
                        wrd = src[r, pl.ds(v * 16, 16)]
                        ev = lax.bitcast_convert_type(
                            jnp.left_shift(wrd, 16), jnp.float32) * bc
                        od = lax.bitcast_convert_type(
                            wrd & himask, jnp.float32) * bc
                        dst[r, pl.ds(v * 32, 16)] = ev
                        dst[r, pl.ds(v * 32 + 16, 16)] = od
                return 0

            lax.fori_loop(0, K // 16, gi_body, 0)

        stage(0, 0)
        wait_stage(0)
        gb0, gs0 = gbufs[0]
        gb1, gs1 = gbufs[1]
        pltpu.async_copy(g_hbm.at[row_r.at[0, 0]], gb0, gs0)
        pltpu.async_copy(g_hbm.at[row_r.at[0, 1]], gb1, gs1)

        def super_body(s2, _):
            for ss in range(2):
                s = s2 * 2 + ss
                slot = ss
                other = 1 - ss
                have_next = s + 1 < nsuper

                def q_body(q2, _):
                    last_pair = q2 == SUP // 2 - 1
                    for t in range(2):
                        q = q2 * 2 + t
                        gbuf, gsem = gbufs[t]
                        sbuf, ssem = sbufs[t]
                        gchunk = s * SUP + q

                        if t == 0:
                            @pl.when(jnp.logical_and(q2 == 1, have_next))
                            def _():
                                stage(s + 1, other)

                            @pl.when(jnp.logical_and(last_pair, have_next))
                            def _():
                                wait_stage(other)

                        pltpu.make_async_copy(
                            g_hbm.at[row_r.at[slot, q]], gbuf, gsem).wait()

                        @pl.when(gchunk >= 2)
                        def _():
                            pltpu.make_async_copy(
                                sbuf, acc.at[col_r.at[slot, q]], ssem).wait()

                        scale_rows(slot, q, gbuf, sbuf)
                        pltpu.async_copy(sbuf, acc.at[col_r.at[slot, q]],
                                         ssem, add=True)

                        @pl.when(jnp.logical_not(last_pair))
                        def _():
                            pltpu.async_copy(
                                g_hbm.at[row_r.at[slot, q + 2]], gbuf, gsem)

                        @pl.when(jnp.logical_and(last_pair, have_next))
                        def _():
                            pltpu.async_copy(
                                g_hbm.at[row_r.at[other, t]], gbuf, gsem)
                    return 0

                lax.fori_loop(0, SUP // 2, q_body, 0)
            return 0

        lax.fori_loop(0, nsuper // 2, super_body, 0)

        pltpu.make_async_copy(sbuf0, acc.at[col_r.at[0, 0]], ssem0).wait()
        pltpu.make_async_copy(sbuf1, acc.at[col_r.at[0, 1]], ssem1).wait()

        plsc.subcore_barrier()
        pltpu.sync_copy(
            acc.at[pl.ds(sid * rows_per_tile, rows_per_tile)],
            out_hbm.at[cid, pl.ds(sid * rows_per_tile, rows_per_tile)])

    return hop_kernel


def _pack_pairs(g, c):
    parts = []
    for v in range(c // 32):
        x1 = g[:, v * 32:v * 32 + 16].astype(jnp.bfloat16)
        x2 = g[:, v * 32 + 16:v * 32 + 32].astype(jnp.bfloat16)
        u1 = lax.bitcast_convert_type(x1, jnp.uint16).astype(jnp.int32)
        u2 = lax.bitcast_convert_type(x2, jnp.uint16).astype(jnp.int32)
        parts.append(u1 | (u2 << 16))
    return jnp.concatenate(parts, axis=1)


def _tc_prep(dp_t, x_pad, n_pad, c, blk):
    def body(dp_ref, x_ref, g0_ref, g0h_ref, dis_ref, d1_ref):
        dp = dp_ref[...]
        deg = dp[:, 0:1] + dp[:, 1:2] + 1.0
        dis = lax.rsqrt(deg)
        g0 = x_ref[...] * dis
        g0_ref[...] = g0
        g0h_ref[...] = _pack_pairs(g0, c)
        dis_ref[...] = dis
        d1_ref[...] = 1.0 / deg

    grid = n_pad // blk
    return pl.pallas_call(
        body,
        grid=(grid,),
        in_specs=[
            pl.BlockSpec((blk, NC), lambda i: (i, 0)),
            pl.BlockSpec((blk, c), lambda i: (i, 0)),
        ],
        out_specs=[
            pl.BlockSpec((blk, c), lambda i: (i, 0)),
            pl.BlockSpec((blk, c // 2), lambda i: (i, 0)),
            pl.BlockSpec((blk, 1), lambda i: (i, 0)),
            pl.BlockSpec((blk, 1), lambda i: (i, 0)),
        ],
        out_shape=[
            jax.ShapeDtypeStruct((n_pad, c), jnp.float32),
            jax.ShapeDtypeStruct((n_pad, c // 2), jnp.int32),
            jax.ShapeDtypeStruct((n_pad, 1), jnp.float32),
            jax.ShapeDtypeStruct((n_pad, 1), jnp.float32),
        ],
    )(dp_t, x_pad)


def _tc_mid(s1a, s1b, g0, d1, n_pad, c, blk):
    def body(a_ref, b_ref, g0_ref, d1_ref, g1_ref, g1h_ref):
        g1 = d1_ref[...] * (a_ref[...] + b_ref[...] + g0_ref[...])
        g1_ref[...] = g1
        g1h_ref[...] = _pack_pairs(g1, g1.shape[1])

    grid = n_pad // blk
    return pl.pallas_call(
        body,
        grid=(grid,),
        in_specs=[
            pl.BlockSpec((blk, c), lambda i: (i, 0)),
            pl.BlockSpec((blk, c), lambda i: (i, 0)),
            pl.BlockSpec((blk, c), lambda i: (i, 0)),
            pl.BlockSpec((blk, 1), lambda i: (i, 0)),
        ],
        out_specs=[
            pl.BlockSpec((blk, c), lambda i: (i, 0)),
            pl.BlockSpec((blk, c // 2), lambda i: (i, 0)),
        ],
        out_shape=[
            jax.ShapeDtypeStruct((n_pad, c), jnp.float32),
            jax.ShapeDtypeStruct((n_pad, c // 2), jnp.int32),
        ],
    )(s1a, s1b, g0, d1)


def _tc_final(W, b2, pa2, gm2, bt2, s2a, s2b, g1, dis, bat, n, n_pad, c, blk):
    grid = n_pad // blk
    inv_n = 1.0 / n

    def body(W_ref, b_ref, pa_ref, gm_ref, bt_ref, a_ref, bb_ref, g1_ref,
             dis_ref, bat_ref, out_ref, seg_sum, seg_max, seg_min, cnt,
             ssum, ssq):
        i = pl.program_id(0)

        @pl.when(i == 0)
        def _():
            seg_sum[...] = jnp.zeros_like(seg_sum)
            seg_max[...] = jnp.full_like(seg_max, -jnp.inf)
            seg_min[...] = jnp.full_like(seg_min, jnp.inf)
            cnt[...] = jnp.zeros_like(cnt)
            ssum[...] = jnp.zeros_like(ssum)
            ssq[...] = jnp.zeros_like(ssq)

        h2 = dis_ref[...] * (a_ref[...] + bb_ref[...] + g1_ref[...])
        z = lax.dot_general(
            h2, W_ref[...], (((1,), (1,)), ((), ())),
            precision=lax.Precision.HIGHEST,
            preferred_element_type=jnp.float32) + b_ref[...]
        pa = pa_ref[...]
        z = jnp.where(z >= 0, z, pa * z)

        bat_blk = bat_ref[...]
        gids = lax.broadcasted_iota(jnp.int32, (1, G), 1)
        mask = bat_blk == gids
        maskf = mask.astype(jnp.float32)
        valid = bat_blk < G
        zv = jnp.where(valid, z, 0.0)

        seg_sum[...] += lax.dot_general(
            maskf, z, (((0,), (0,)), ((), ())),
            precision=lax.Precision.HIGHEST,
            preferred_element_type=jnp.float32)
        cnt[...] += lax.dot_general(
            maskf, jnp.ones_like(z), (((0,), (0,)), ((), ())),
            precision=lax.Precision.HIGHEST,
            preferred_element_type=jnp.float32)
        ssum[...] += jnp.sum(zv, axis=0, keepdims=True)
        ssq[...] += jnp.sum(zv * zv, axis=0, keepdims=True)
        for g in range(G):
            mg = mask[:, g:g + 1]
            zmax = jnp.max(jnp.where(mg, z, -jnp.inf), axis=0, keepdims=True)
            zmin = jnp.min(jnp.where(mg, z, jnp.inf), axis=0, keepdims=True)
            seg_max[g:g + 1, :] = jnp.maximum(seg_max[g:g + 1, :], zmax)
            seg_min[g:g + 1, :] = jnp.minimum(seg_min[g:g + 1, :], zmin)

        @pl.when(i == grid - 1)
        def _():
            mean = ssum[...] * inv_n
            var = ssq[...] * inv_n - mean * mean
            scale = gm_ref[...] * lax.rsqrt(var + 1e-5)
            cc = cnt[...]
            nonempty = cc > 0
            sm = seg_sum[...] / jnp.maximum(cc, 1.0)
            om = (sm - mean) * scale + bt_ref[...]
            mx = jnp.where(gm_ref[...] >= 0, seg_max[...], seg_min[...])
            ox = (mx - mean) * scale + bt_ref[...]
            out_ref[:, :c] = jnp.where(nonempty, om, 0.0)
            out_ref[:, c:] = jnp.where(nonempty, ox, 0.0)

    return pl.pallas_call(
        body,
        grid=(grid,),
        in_specs=[
            pl.BlockSpec((c, c), lambda i: (0, 0)),
            pl.BlockSpec((1, c), lambda i: (0, 0)),
            pl.BlockSpec((1, c), lambda i: (0, 0)),
            pl.BlockSpec((1, c), lambda i: (0, 0)),
            pl.BlockSpec((1, c), lambda i: (0, 0)),
            pl.BlockSpec((blk, c), lambda i: (i, 0)),
            pl.BlockSpec((blk, c), lambda i: (i, 0)),
            pl.BlockSpec((blk, c), lambda i: (i, 0)),
            pl.BlockSpec((blk, 1), lambda i: (i, 0)),
            pl.BlockSpec((blk, 1), lambda i: (i, 0)),
        ],
        out_specs=pl.BlockSpec((G, 2 * c), lambda i: (0, 0)),
        out_shape=jax.ShapeDtypeStruct((G, 2 * c), jnp.float32),
        scratch_shapes=[
            pltpu.VMEM((G, c), jnp.float32),
            pltpu.VMEM((G, c), jnp.float32),
            pltpu.VMEM((G, c), jnp.float32),
            pltpu.VMEM((G, c), jnp.float32),
            pltpu.VMEM((1, c), jnp.float32),
            pltpu.VMEM((1, c), jnp.float32),
        ],
    )(W, b2, pa2, gm2, bt2, s2a, s2b, g1, dis, bat)


def kernel(x, edge_index, edge_weight, batch, W, b, prelu_a, gamma, beta):
    n, c = x.shape
    e = edge_weight.shape[0]

    sup_edges = SUP * K
    t_pair = -(-e // (NS * sup_edges))
    t_pair += t_pair % 2
    frac1 = 0.45
    n1 = max(2, int(round(t_pair * frac1 / 2)) * 2)
    n0 = t_pair - n1
    e_pad = NS * t_pair * sup_edges
    c0 = n0 * SUP
    c1 = n1 * SUP

    row = edge_index[0]
    col = edge_index[1]
    pad_e = e_pad - e
    row_f = jnp.concatenate([row, jnp.zeros((pad_e,), jnp.int32)])
    col_f = jnp.concatenate([col, jnp.zeros((pad_e,), jnp.int32)])
    w_f = jnp.concatenate([edge_weight, jnp.zeros((pad_e,), jnp.float32)])
    row_s = row_f.reshape(-1, SUP, K)
    col_s = col_f.reshape(-1, SUP, K)
    w_s = w_f.reshape(-1, SUP, K)
    col_p = col_f.reshape(-1, K)
    w_p = w_f.reshape(-1, K)

    n_pad = -(-n // 128) * 128
    x_pad = jnp.pad(x, ((0, n_pad - n), (0, 0)))
    bat_pad = jnp.pad(batch, (0, n_pad - n),
                      constant_values=G).reshape(n_pad, 1)

    n_pad_deg = -(-n // (NS * 16)) * (NS * 16)
    deg_part = _make_sc_deg(n_pad_deg, c0, c1)(col_p, w_p)
    dp_t = deg_part[:, :n_pad].T

    blk = n_pad // 8
    g0, g0h, dis, d1 = _tc_prep(dp_t, x_pad, n_pad, c, blk)

    hop = _make_sc_hop(n_pad, c, n0, n1)
    s1 = hop(g0h, row_s, col_s, w_s)
    g1, g1h = _tc_mid(s1[0], s1[1], g0, d1, n_pad, c, blk)
    s2 = hop(g1h, row_s, col_s, w_s)

    b2 = b.reshape(1, c)
    pa2 = prelu_a.reshape(1, c)
    gm2 = gamma.reshape(1, c)
    bt2 = beta.reshape(1, c)
    return _tc_final(W, b2, pa2, gm2, bt2, s2[0], s2[1], g1, dis, bat_pad,
                     n, n_pad, c, blk)
